# Initial kernel scaffold; baseline (speedup 1.0000x reference)
import jax
import jax.numpy as jnp
from jax.experimental import pallas as pl


def kernel(Seq, previous_pred):
    raise NotImplementedError("write your pallas kernel here")


# speedup vs baseline: 2.6314x; 2.6314x over previous
import jax
import jax.numpy as jnp
from jax import lax
from jax.experimental import pallas as pl
from jax.experimental.pallas import tpu as pltpu
from jax.experimental.pallas import tpu_sc as plsc

L_EV = 2_000_000
CHUNK = 8192
VPC = CHUNK // 16
NCH_FULL = L_EV // CHUNK
TAIL = L_EV - NCH_FULL * CHUNK
TAIL_V = TAIL // 16
NW = 32
EXTRA = NCH_FULL - 7 * NW
SLAB = 8 * CHUNK
NQ = 10_000
QPW = 320
QV = QPW // 16

_mesh = plsc.VectorSubcoreMesh(
    core_axis_name="c", subcore_axis_name="s", num_cores=2, num_subcores=16)


def _stage_a_body(seq_hbm, box_hbm, counts_hbm, slabs_hbm,
                  xbuf, ybuf, idxbuf, boxv, cntv):
    wid = lax.axis_index("c") * 16 + lax.axis_index("s")
    pltpu.sync_copy(box_hbm, boxv)
    c0 = 7 * wid + jnp.minimum(wid, EXTRA)
    nch = jnp.where((wid < EXTRA) | (wid == NW - 1), 8, 7)
    cntv[...] = jnp.zeros((16,), jnp.int32)

    def chunk_body(ci, carry):
        is_tail = (wid == NW - 1) & (ci == 7)
        base = jnp.where(is_tail, L_EV - CHUNK, (c0 + ci) * CHUNK)
        vlo = jnp.where(is_tail, VPC - TAIL_V, 0)
        pltpu.sync_copy(seq_hbm.at[pl.ds(base, CHUNK)], xbuf)
        pltpu.sync_copy(seq_hbm.at[pl.ds(L_EV + base, CHUNK)], ybuf)

        def vreg_body(vi, c2):
            xv = xbuf[pl.ds(vi * 16, 16)]
            yv = ybuf[pl.ds(vi * 16, 16)]
            m = ((xv >= boxv[0, :]) & (xv <= boxv[2, :])
                 & (yv >= boxv[1, :]) & (yv <= boxv[3, :]))
            mi = jnp.where(m, 1, 0)
            cv = cntv[...]
            pos = cv + plsc.cumsum(mi) - 1
            idxv = base + vi * 16 + lax.iota(jnp.int32, 16)
            plsc.store_scatter(idxbuf, [pos], idxv, mask=m)
            cntv[...] = cv + plsc.all_reduce_population_count(m)
            return c2

        return lax.fori_loop(vlo, VPC, vreg_body, carry)

    lax.fori_loop(0, nch, chunk_body, 0)
    pltpu.sync_copy(cntv, counts_hbm.at[wid])
    pltpu.sync_copy(idxbuf, slabs_hbm.at[wid])


_STAGE_A = pl.kernel(
    _stage_a_body,
    out_type=(
        jax.ShapeDtypeStruct((NW, 16), jnp.int32),
        jax.ShapeDtypeStruct((NW, SLAB), jnp.int32),
    ),
    mesh=_mesh,
    scratch_types=(
        pltpu.VMEM((CHUNK,), jnp.float32),
        pltpu.VMEM((CHUNK,), jnp.float32),
        pltpu.VMEM((SLAB,), jnp.int32),
        pltpu.VMEM((4, 16), jnp.float32),
        pltpu.VMEM((16,), jnp.int32),
    ),
    compiler_params=pltpu.CompilerParams(needs_layout_passes=False),
)


def _stage_b_body(seq_hbm, slabs_hbm, counts_hbm, pn_hbm, prm_hbm, out_hbm,
                  cntbuf, inclbuf, exbuf, pnbuf, origbuf, valbuf, prmbuf, dsem):
    wid = lax.axis_index("c") * 16 + lax.axis_index("s")
    qstart = jnp.minimum(wid * QPW, NQ - QPW)
    pltpu.sync_copy(counts_hbm, cntbuf)
    pltpu.sync_copy(pn_hbm.at[pl.ds(qstart, QPW)], pnbuf)
    pltpu.sync_copy(prm_hbm, prmbuf)
    lanes = lax.iota(jnp.int32, 16)
    zeros = jnp.zeros((16,), jnp.int32)
    c_lo = plsc.load_gather(cntbuf, [lanes, zeros])
    c_hi = plsc.load_gather(cntbuf, [lanes + 16, zeros])
    incl_lo = plsc.cumsum(c_lo)
    inclbuf[pl.ds(0, 16)] = incl_lo
    t_lo = jnp.sum(c_lo)
    incl_hi = plsc.cumsum(c_hi) + t_lo
    inclbuf[pl.ds(16, 16)] = incl_hi
    exbuf[pl.ds(0, 16)] = incl_lo - c_lo
    exbuf[pl.ds(16, 16)] = incl_hi - c_hi

    def rank_body(qv, carry):
        r = pnbuf[pl.ds(qv * 16, 16)]
        w = jnp.zeros((16,), jnp.int32)
        for bit in (16, 8, 4, 2, 1):
            ik = plsc.load_gather(inclbuf, [w + (bit - 1)])
            w = w + jnp.where(r >= ik, bit, 0)
        lr = r - plsc.load_gather(exbuf, [w])
        sidx = w * SLAB + lr
        pltpu.async_copy(slabs_hbm.at[sidx], origbuf.at[pl.ds(qv * 16, 16)], dsem)
        return carry

    lax.fori_loop(0, QV, rank_body, 0)
    pltpu.make_async_copy(slabs_hbm.at[pl.ds(0, QPW)], origbuf, dsem).wait()

    def gat_body(j, carry):
        c = j // QV
        rr = j - c * QV
        ov = origbuf[pl.ds(rr * 16, 16)]
        cidx = ov + c * L_EV
        pltpu.async_copy(seq_hbm.at[cidx], valbuf.at[pl.ds(j * 16, 16)], dsem)
        return carry

    lax.fori_loop(0, 5 * QV, gat_body, 0)
    pltpu.make_async_copy(seq_hbm.at[pl.ds(0, 5 * QPW)], valbuf, dsem).wait()

    for ch in range(2):
        lov = prmbuf[ch, :]
        dv = prmbuf[2 + ch, :]
        for rr in range(QV):
            j = ch * QV + rr
            v = valbuf[pl.ds(j * 16, 16)]
            valbuf[pl.ds(j * 16, 16)] = (v - lov) / dv

    for c in range(5):
        pltpu.sync_copy(valbuf.at[pl.ds(c * QPW, QPW)],
                        out_hbm.at[pl.ds(c * NQ + qstart, QPW)])


_STAGE_B = pl.kernel(
    _stage_b_body,
    out_type=jax.ShapeDtypeStruct((5 * NQ,), jnp.float32),
    mesh=_mesh,
    scratch_types=(
        pltpu.VMEM((NW, 16), jnp.int32),
        pltpu.VMEM((NW,), jnp.int32),
        pltpu.VMEM((NW,), jnp.int32),
        pltpu.VMEM((QPW,), jnp.int32),
        pltpu.VMEM((QPW,), jnp.int32),
        pltpu.VMEM((5 * QPW,), jnp.float32),
        pltpu.VMEM((4, 16), jnp.float32),
        pltpu.SemaphoreType.DMA,
    ),
    compiler_params=pltpu.CompilerParams(needs_layout_passes=False),
)


def kernel(Seq, previous_pred):
    lo = jnp.clip(previous_pred[:2] - previous_pred[2:] / 2 - 0.25, 0.0, 1.0)
    hi = jnp.clip(lo + previous_pred[2:] + 0.5, 0.0, 1.0)
    lo = jnp.clip(hi - previous_pred[2:] - 0.5, 0.0, 1.0)
    seq_flat = jnp.reshape(Seq, (-1,))
    box = jnp.stack([lo[0], lo[1], hi[0], hi[1]])
    box_b = jnp.broadcast_to(box[:, None], (4, 16))
    counts, slabs = _STAGE_A(seq_flat, box_b)
    n_total = jnp.sum(counts[:, 0])
    pn = jax.random.randint(jax.random.key(1), (NQ,), 0, n_total)
    d0 = hi[0] - lo[0] + 1e-6
    d1 = hi[1] - lo[1] + 1e-6
    prm = jnp.stack([lo[0], lo[1], d0, d1])
    prm_b = jnp.broadcast_to(prm[:, None], (4, 16))
    out = _STAGE_B(seq_flat, jnp.reshape(slabs, (-1,)), counts,
                   pn.astype(jnp.int32), prm_b)
    return jnp.reshape(out, (1, 5, NQ)), lo, hi


# speedup vs baseline: 2.6442x; 1.0049x over previous
import jax
import jax.numpy as jnp
from jax import lax
from jax.experimental import pallas as pl
from jax.experimental.pallas import tpu as pltpu
from jax.experimental.pallas import tpu_sc as plsc

L_EV = 2_000_000
CHUNK = 8192
VPC = CHUNK // 16
NCH_FULL = L_EV // CHUNK
TAIL = L_EV - NCH_FULL * CHUNK
TAIL_V = TAIL // 16
NW = 32
EXTRA = NCH_FULL - 7 * NW
SLAB = 8 * CHUNK
NQ = 10_000
QPW = 320
QV = QPW // 16

_mesh = plsc.VectorSubcoreMesh(
    core_axis_name="c", subcore_axis_name="s", num_cores=2, num_subcores=16)


def _stage_a_body(seq_hbm, box_hbm, counts_hbm, slabs_hbm,
                  xbuf, ybuf, idxbuf, boxv, cntv):
    wid = lax.axis_index("c") * 16 + lax.axis_index("s")
    pltpu.sync_copy(box_hbm, boxv)
    c0 = 7 * wid + jnp.minimum(wid, EXTRA)
    nch = jnp.where((wid < EXTRA) | (wid == NW - 1), 8, 7)
    cntv[...] = jnp.zeros((16,), jnp.int32)

    def chunk_body(ci, carry):
        is_tail = (wid == NW - 1) & (ci == 7)
        base = jnp.where(is_tail, L_EV - CHUNK, (c0 + ci) * CHUNK)
        vlo = jnp.where(is_tail, VPC - TAIL_V, 0)
        pltpu.sync_copy(seq_hbm.at[pl.ds(base, CHUNK)], xbuf)
        pltpu.sync_copy(seq_hbm.at[pl.ds(L_EV + base, CHUNK)], ybuf)

        def vreg_body(vi, c2):
            xv = xbuf[pl.ds(vi * 16, 16)]
            yv = ybuf[pl.ds(vi * 16, 16)]
            m = ((xv >= boxv[0, :]) & (xv <= boxv[2, :])
                 & (yv >= boxv[1, :]) & (yv <= boxv[3, :]))
            mi = jnp.where(m, 1, 0)
            cv = cntv[...]
            pos = cv + plsc.cumsum(mi) - 1
            idxv = base + vi * 16 + lax.iota(jnp.int32, 16)
            plsc.store_scatter(idxbuf, [pos], idxv, mask=m)
            cntv[...] = cv + plsc.all_reduce_population_count(m)
            return c2

        return lax.fori_loop(vlo, VPC, vreg_body, carry)

    lax.fori_loop(0, nch, chunk_body, 0)
    pltpu.sync_copy(cntv, counts_hbm.at[wid])
    pltpu.sync_copy(idxbuf, slabs_hbm.at[pl.ds(wid * SLAB, SLAB)])


_STAGE_A = pl.kernel(
    _stage_a_body,
    out_type=(
        jax.ShapeDtypeStruct((NW, 16), jnp.int32),
        jax.ShapeDtypeStruct((NW * SLAB,), jnp.int32),
    ),
    mesh=_mesh,
    scratch_types=(
        pltpu.VMEM((CHUNK,), jnp.float32),
        pltpu.VMEM((CHUNK,), jnp.float32),
        pltpu.VMEM((SLAB,), jnp.int32),
        pltpu.VMEM((4, 16), jnp.float32),
        pltpu.VMEM((16,), jnp.int32),
    ),
    compiler_params=pltpu.CompilerParams(needs_layout_passes=False),
)


def _stage_b_body(seq_hbm, slabs_hbm, counts_hbm, pn_hbm, prm_hbm, out_hbm,
                  cntbuf, inclbuf, exbuf, pnbuf, origbuf, valbuf, prmbuf, dsem):
    wid = lax.axis_index("c") * 16 + lax.axis_index("s")
    qstart = jnp.minimum(wid * QPW, NQ - QPW)
    pltpu.sync_copy(counts_hbm, cntbuf)
    pltpu.sync_copy(pn_hbm.at[pl.ds(qstart, QPW)], pnbuf)
    pltpu.sync_copy(prm_hbm, prmbuf)
    lanes = lax.iota(jnp.int32, 16)
    zeros = jnp.zeros((16,), jnp.int32)
    c_lo = plsc.load_gather(cntbuf, [lanes, zeros])
    c_hi = plsc.load_gather(cntbuf, [lanes + 16, zeros])
    incl_lo = plsc.cumsum(c_lo)
    inclbuf[pl.ds(0, 16)] = incl_lo
    t_lo = jnp.sum(c_lo)
    incl_hi = plsc.cumsum(c_hi) + t_lo
    inclbuf[pl.ds(16, 16)] = incl_hi
    exbuf[pl.ds(0, 16)] = incl_lo - c_lo
    exbuf[pl.ds(16, 16)] = incl_hi - c_hi

    def rank_body(qv, carry):
        r = pnbuf[pl.ds(qv * 16, 16)]
        w = jnp.zeros((16,), jnp.int32)
        for bit in (16, 8, 4, 2, 1):
            ik = plsc.load_gather(inclbuf, [w + (bit - 1)])
            w = w + jnp.where(r >= ik, bit, 0)
        lr = r - plsc.load_gather(exbuf, [w])
        sidx = w * SLAB + lr
        pltpu.async_copy(slabs_hbm.at[sidx], origbuf.at[pl.ds(qv * 16, 16)], dsem)
        return carry

    lax.fori_loop(0, QV, rank_body, 0)
    pltpu.make_async_copy(slabs_hbm.at[pl.ds(0, QPW)], origbuf, dsem).wait()

    def gat_body(j, carry):
        c = j // QV
        rr = j - c * QV
        ov = origbuf[pl.ds(rr * 16, 16)]
        cidx = ov + c * L_EV
        pltpu.async_copy(seq_hbm.at[cidx], valbuf.at[pl.ds(j * 16, 16)], dsem)
        return carry

    lax.fori_loop(0, 5 * QV, gat_body, 0)
    pltpu.make_async_copy(seq_hbm.at[pl.ds(0, 5 * QPW)], valbuf, dsem).wait()

    for ch in range(2):
        lov = prmbuf[ch, :]
        dv = prmbuf[2 + ch, :]
        for rr in range(QV):
            j = ch * QV + rr
            v = valbuf[pl.ds(j * 16, 16)]
            valbuf[pl.ds(j * 16, 16)] = (v - lov) / dv

    for c in range(5):
        pltpu.sync_copy(valbuf.at[pl.ds(c * QPW, QPW)],
                        out_hbm.at[pl.ds(c * NQ + qstart, QPW)])


_STAGE_B = pl.kernel(
    _stage_b_body,
    out_type=jax.ShapeDtypeStruct((5 * NQ,), jnp.float32),
    mesh=_mesh,
    scratch_types=(
        pltpu.VMEM((NW, 16), jnp.int32),
        pltpu.VMEM((NW,), jnp.int32),
        pltpu.VMEM((NW,), jnp.int32),
        pltpu.VMEM((QPW,), jnp.int32),
        pltpu.VMEM((QPW,), jnp.int32),
        pltpu.VMEM((5 * QPW,), jnp.float32),
        pltpu.VMEM((4, 16), jnp.float32),
        pltpu.SemaphoreType.DMA,
    ),
    compiler_params=pltpu.CompilerParams(needs_layout_passes=False),
)


def kernel(Seq, previous_pred):
    lo = jnp.clip(previous_pred[:2] - previous_pred[2:] / 2 - 0.25, 0.0, 1.0)
    hi = jnp.clip(lo + previous_pred[2:] + 0.5, 0.0, 1.0)
    lo = jnp.clip(hi - previous_pred[2:] - 0.5, 0.0, 1.0)
    seq_flat = jnp.reshape(Seq, (-1,))
    box = jnp.stack([lo[0], lo[1], hi[0], hi[1]])
    box_b = jnp.broadcast_to(box[:, None], (4, 16))
    counts, slabs = _STAGE_A(seq_flat, box_b)
    n_total = jnp.sum(counts[:, 0])
    pn = jax.random.randint(jax.random.key(1), (NQ,), 0, n_total)
    d0 = hi[0] - lo[0] + 1e-6
    d1 = hi[1] - lo[1] + 1e-6
    prm = jnp.stack([lo[0], lo[1], d0, d1])
    prm_b = jnp.broadcast_to(prm[:, None], (4, 16))
    out = _STAGE_B(seq_flat, slabs, counts, pn.astype(jnp.int32), prm_b)
    return jnp.reshape(out, (1, 5, NQ)), lo, hi


# speedup vs baseline: 2.7271x; 1.0314x over previous
import jax
import jax.numpy as jnp
from jax import lax
from jax.experimental import pallas as pl
from jax.experimental.pallas import tpu as pltpu
from jax.experimental.pallas import tpu_sc as plsc

L_EV = 2_000_000
CHUNK = 8192
VPC = CHUNK // 16
NCH_FULL = L_EV // CHUNK
TAIL = L_EV - NCH_FULL * CHUNK
TAIL_V = TAIL // 16
NW = 32
EXTRA = NCH_FULL - 7 * NW
SLAB = 8 * CHUNK
NQ = 10_000
QPW = 320
QV = QPW // 16

_mesh = plsc.VectorSubcoreMesh(
    core_axis_name="c", subcore_axis_name="s", num_cores=2, num_subcores=16)


def _stage_a_body(seq_hbm, box_hbm, counts_hbm, slabs_hbm,
                  xbuf, ybuf, idxbuf, boxv, cntv):
    wid = lax.axis_index("c") * 16 + lax.axis_index("s")
    pltpu.sync_copy(box_hbm, boxv)
    c0 = 7 * wid + jnp.minimum(wid, EXTRA)
    nch = jnp.where((wid < EXTRA) | (wid == NW - 1), 8, 7)
    xlo = boxv[0, :]
    ylo = boxv[1, :]
    xhi = boxv[2, :]
    yhi = boxv[3, :]
    lanes = lax.iota(jnp.int32, 16)

    def chunk_body(ci, cnt_vec):
        is_tail = (wid == NW - 1) & (ci == 7)
        base = jnp.where(is_tail, L_EV - CHUNK, (c0 + ci) * CHUNK)
        vlo = jnp.where(is_tail, VPC - TAIL_V, 0)
        pltpu.sync_copy(seq_hbm.at[pl.ds(base, CHUNK)], xbuf)
        pltpu.sync_copy(seq_hbm.at[pl.ds(L_EV + base, CHUNK)], ybuf)

        @plsc.parallel_loop(vlo, VPC, step=1, unroll=4, carry=cnt_vec)
        def vreg_body(vi, cv):
            xv = xbuf[pl.ds(vi * 16, 16)]
            yv = ybuf[pl.ds(vi * 16, 16)]
            m = (xv >= xlo) & (xv <= xhi) & (yv >= ylo) & (yv <= yhi)
            mi = jnp.where(m, 1, 0)
            pos = cv + plsc.cumsum(mi) - 1
            idxv = (base + vi * 16) + lanes
            plsc.store_scatter(idxbuf, [pos], idxv, mask=m)
            return cv + plsc.all_reduce_population_count(m)

        return vreg_body

    cnt_vec = lax.fori_loop(0, nch, chunk_body, jnp.zeros((16,), jnp.int32))
    cntv[...] = cnt_vec
    pltpu.sync_copy(cntv, counts_hbm.at[wid])
    pltpu.sync_copy(idxbuf, slabs_hbm.at[pl.ds(wid * SLAB, SLAB)])


_STAGE_A = pl.kernel(
    _stage_a_body,
    out_type=(
        jax.ShapeDtypeStruct((NW, 16), jnp.int32),
        jax.ShapeDtypeStruct((NW * SLAB,), jnp.int32),
    ),
    mesh=_mesh,
    scratch_types=(
        pltpu.VMEM((CHUNK,), jnp.float32),
        pltpu.VMEM((CHUNK,), jnp.float32),
        pltpu.VMEM((SLAB,), jnp.int32),
        pltpu.VMEM((4, 16), jnp.float32),
        pltpu.VMEM((16,), jnp.int32),
    ),
    compiler_params=pltpu.CompilerParams(needs_layout_passes=False),
)


def _stage_b_body(seq_hbm, slabs_hbm, counts_hbm, pn_hbm, prm_hbm, out_hbm,
                  cntbuf, inclbuf, exbuf, pnbuf, origbuf, valbuf, prmbuf, dsem):
    wid = lax.axis_index("c") * 16 + lax.axis_index("s")
    qstart = jnp.minimum(wid * QPW, NQ - QPW)
    pltpu.sync_copy(counts_hbm, cntbuf)
    pltpu.sync_copy(pn_hbm.at[pl.ds(qstart, QPW)], pnbuf)
    pltpu.sync_copy(prm_hbm, prmbuf)
    lanes = lax.iota(jnp.int32, 16)
    zeros = jnp.zeros((16,), jnp.int32)
    c_lo = plsc.load_gather(cntbuf, [lanes, zeros])
    c_hi = plsc.load_gather(cntbuf, [lanes + 16, zeros])
    incl_lo = plsc.cumsum(c_lo)
    inclbuf[pl.ds(0, 16)] = incl_lo
    t_lo = jnp.sum(c_lo)
    incl_hi = plsc.cumsum(c_hi) + t_lo
    inclbuf[pl.ds(16, 16)] = incl_hi
    exbuf[pl.ds(0, 16)] = incl_lo - c_lo
    exbuf[pl.ds(16, 16)] = incl_hi - c_hi

    def rank_body(qv, carry):
        r = pnbuf[pl.ds(qv * 16, 16)]
        w = jnp.zeros((16,), jnp.int32)
        for bit in (16, 8, 4, 2, 1):
            ik = plsc.load_gather(inclbuf, [w + (bit - 1)])
            w = w + jnp.where(r >= ik, bit, 0)
        lr = r - plsc.load_gather(exbuf, [w])
        sidx = w * SLAB + lr
        pltpu.async_copy(slabs_hbm.at[sidx], origbuf.at[pl.ds(qv * 16, 16)], dsem)
        return carry

    lax.fori_loop(0, QV, rank_body, 0)
    pltpu.make_async_copy(slabs_hbm.at[pl.ds(0, QPW)], origbuf, dsem).wait()

    def gat_body(j, carry):
        c = j // QV
        rr = j - c * QV
        ov = origbuf[pl.ds(rr * 16, 16)]
        cidx = ov + c * L_EV
        pltpu.async_copy(seq_hbm.at[cidx], valbuf.at[pl.ds(j * 16, 16)], dsem)
        return carry

    lax.fori_loop(0, 5 * QV, gat_body, 0)
    pltpu.make_async_copy(seq_hbm.at[pl.ds(0, 5 * QPW)], valbuf, dsem).wait()

    for ch in range(2):
        lov = prmbuf[ch, :]
        dv = prmbuf[2 + ch, :]
        for rr in range(QV):
            j = ch * QV + rr
            v = valbuf[pl.ds(j * 16, 16)]
            valbuf[pl.ds(j * 16, 16)] = (v - lov) / dv

    for c in range(5):
        pltpu.sync_copy(valbuf.at[pl.ds(c * QPW, QPW)],
                        out_hbm.at[pl.ds(c * NQ + qstart, QPW)])


_STAGE_B = pl.kernel(
    _stage_b_body,
    out_type=jax.ShapeDtypeStruct((5 * NQ,), jnp.float32),
    mesh=_mesh,
    scratch_types=(
        pltpu.VMEM((NW, 16), jnp.int32),
        pltpu.VMEM((NW,), jnp.int32),
        pltpu.VMEM((NW,), jnp.int32),
        pltpu.VMEM((QPW,), jnp.int32),
        pltpu.VMEM((QPW,), jnp.int32),
        pltpu.VMEM((5 * QPW,), jnp.float32),
        pltpu.VMEM((4, 16), jnp.float32),
        pltpu.SemaphoreType.DMA,
    ),
    compiler_params=pltpu.CompilerParams(needs_layout_passes=False),
)


def kernel(Seq, previous_pred):
    lo = jnp.clip(previous_pred[:2] - previous_pred[2:] / 2 - 0.25, 0.0, 1.0)
    hi = jnp.clip(lo + previous_pred[2:] + 0.5, 0.0, 1.0)
    lo = jnp.clip(hi - previous_pred[2:] - 0.5, 0.0, 1.0)
    seq_flat = jnp.reshape(Seq, (-1,))
    box = jnp.stack([lo[0], lo[1], hi[0], hi[1]])
    box_b = jnp.broadcast_to(box[:, None], (4, 16))
    counts, slabs = _STAGE_A(seq_flat, box_b)
    n_total = jnp.sum(counts[:, 0])
    pn = jax.random.randint(jax.random.key(1), (NQ,), 0, n_total)
    d0 = hi[0] - lo[0] + 1e-6
    d1 = hi[1] - lo[1] + 1e-6
    prm = jnp.stack([lo[0], lo[1], d0, d1])
    prm_b = jnp.broadcast_to(prm[:, None], (4, 16))
    out = _STAGE_B(seq_flat, slabs, counts, pn.astype(jnp.int32), prm_b)
    return jnp.reshape(out, (1, 5, NQ)), lo, hi
